# baseline (speedup 1.0000x reference)
import functools

import jax
import jax.numpy as jnp
from jax import lax
from jax.experimental import pallas as pl
from jax.experimental.pallas import tpu as pltpu
from jax.experimental.pallas import tpu_sc as plsc

B, S, M, H = 1024, 200, 4, 128
N_STATIC = 8
NC, NS = 2, 16
NW = NC * NS
ROWS_PER_W = B // NW
E = 40
CHUNKS = S // E
HC = H // 16
SP = 208


def _sc_body(dyn_hbm, sidx_hbm, t_hbm, tbl_hbm, stbl_hbm, tw_hbm, tb_hbm,
             out_hbm, ridx_v, ridx2_v, out_v, t_v, sidxa_v, sidxb_v,
             srows_v, wtb_v, gsem0, gsem1, osem0, osem1, ssem, tsem, stsem):
    wid = lax.axis_index("s") * NC + lax.axis_index("c")
    b0 = wid * ROWS_PER_W
    gsem = (gsem0, gsem1)
    osem = (osem0, osem1)

    pltpu.sync_copy(tw_hbm.at[0], wtb_v.at[0])
    pltpu.sync_copy(tb_hbm, wtb_v.at[1])
    wch = [wtb_v[0, pl.ds(h * 16, 16)] for h in range(HC)]
    tbch = [wtb_v[1, pl.ds(h * 16, 16)] for h in range(HC)]

    lanes = lax.iota(jnp.int32, 16)

    tcps = [pltpu.async_copy(
        t_hbm.at[pl.ds((b0 + r) * S, S)], t_v.at[pl.ds(r * SP, S)], tsem)
        for r in range(ROWS_PER_W)]

    def stage_static(b, p):
        sidx = sidxa_v if p == 0 else sidxb_v
        pltpu.sync_copy(sidx_hbm.at[b], sidx)
        pltpu.async_copy(stbl_hbm.at[sidx], srows_v.at[p], ssem)

    def wait_static(p):
        pltpu.make_async_copy(
            stbl_hbm.at[pl.ds(0, N_STATIC)], srows_v.at[p], ssem).wait()

    def transpose_row(p):
        for g in range(13):
            rid = lanes + g * 16
            for m in range(M):
                ridx_v[pl.ds(p * M * SP + m * SP + g * 16, 16)] = \
                    plsc.load_gather(
                        ridx2_v, [rid, jnp.full((16,), m, jnp.int32)])

    stage_static(b0, 0)
    pltpu.sync_copy(dyn_hbm.at[b0], ridx2_v.at[pl.ds(0, S)])
    transpose_row(0)
    for cp in tcps:
        cp.wait()

    def wait_gathers(p):
        for _ in range(CHUNKS * M):
            pltpu.make_async_copy(
                tbl_hbm.at[pl.ds(0, E)], out_v.at[0], gsem[p]).wait()

    def wait_outs(p):
        for _ in range(CHUNKS):
            pltpu.make_async_copy(
                out_v.at[0], out_hbm.at[pl.ds(0, E)], osem[p]).wait()

    def when(pred):
        if pred is True:
            return lambda fn: fn()
        return pl.when(pred)

    def do_row(r, not_first, not_first2, not_last, p):
        b = b0 + r
        wait_static(p)
        base = []
        for h in range(HC):
            hs = pl.ds(h * 16, 16)
            acc = srows_v[p, 0, hs]
            for j in range(1, N_STATIC):
                acc = acc + srows_v[p, j, hs]
            base.append(acc * (1.0 / N_STATIC) + tbch[h])

        @when(not_last)
        def _():
            pltpu.async_copy(
                dyn_hbm.at[b + 1], ridx2_v.at[pl.ds(0, S)], stsem)
            stage_static(b + 1, 1 - p)

        @when(not_first2)
        def _():
            wait_outs(p)

        for ci in range(CHUNKS):
            @plsc.parallel_loop(0, E, unroll=4)
            def pre_body(e, ci=ci):
                tvals = t_v[pl.ds(r * SP + ci * E + (e // 16) * 16, 16)]
                t = lax.gather(
                    tvals, jnp.broadcast_to(e % 16, (16, 1)),
                    lax.GatherDimensionNumbers(
                        offset_dims=(), collapsed_slice_dims=(0,),
                        start_index_map=(0,)),
                    (1,), mode=lax.GatherScatterMode.PROMISE_IN_BOUNDS)
                for h in range(HC):
                    out_v[p * CHUNKS + ci, e, pl.ds(h * 16, 16)] = \
                        base[h] + t * wch[h]
        for ci in range(CHUNKS):
            for m in range(M):
                pltpu.async_copy(
                    tbl_hbm.at[ridx_v.at[pl.ds(p * M * SP + m * SP + ci * E, E)]],
                    out_v.at[p * CHUNKS + ci], gsem[p], add=True)
        @when(not_first)
        def _():
            wait_gathers(1 - p)
            for ci in range(CHUNKS):
                pltpu.async_copy(
                    out_v.at[(1 - p) * CHUNKS + ci],
                    out_hbm.at[pl.ds((b - 1) * S + ci * E, E)], osem[1 - p])

        @when(not_last)
        def _():
            pltpu.make_async_copy(
                dyn_hbm.at[b], ridx2_v.at[pl.ds(0, S)], stsem).wait()
            transpose_row(1 - p)

    def pair_body(i, _):
        r0 = 2 * i
        do_row(r0, i > 0, i > 0, True, 0)
        do_row(r0 + 1, True, i > 0, i < ROWS_PER_W // 2 - 1, 1)
        return 0

    lax.fori_loop(0, ROWS_PER_W // 2, pair_body, 0)

    blast = b0 + ROWS_PER_W - 1
    wait_gathers(1)
    for ci in range(CHUNKS):
        pltpu.async_copy(
            out_v.at[CHUNKS + ci],
            out_hbm.at[pl.ds(blast * S + ci * E, E)], osem[1])
    wait_outs(0)
    wait_outs(1)


@jax.jit
def _run(dyn_idx, static_idx, time, data_tbl, static_tbl, time_w, time_b):
    mesh = plsc.VectorSubcoreMesh(core_axis_name="c", subcore_axis_name="s")
    kfn = functools.partial(
        pl.kernel,
        mesh=mesh,
        out_type=jax.ShapeDtypeStruct((B * S, H), jnp.float32),
        compiler_params=pltpu.CompilerParams(needs_layout_passes=False),
        scratch_types=[
            pltpu.VMEM((2 * M * SP,), jnp.int32),
            pltpu.VMEM((SP, M), jnp.int32),
            pltpu.VMEM((2 * CHUNKS, E, H), jnp.float32),
            pltpu.VMEM((ROWS_PER_W * SP,), jnp.float32),
            pltpu.VMEM((N_STATIC,), jnp.int32),
            pltpu.VMEM((N_STATIC,), jnp.int32),
            pltpu.VMEM((2, N_STATIC, H), jnp.float32),
            pltpu.VMEM((2, H), jnp.float32),
            pltpu.SemaphoreType.DMA,
            pltpu.SemaphoreType.DMA,
            pltpu.SemaphoreType.DMA,
            pltpu.SemaphoreType.DMA,
            pltpu.SemaphoreType.DMA,
            pltpu.SemaphoreType.DMA,
            pltpu.SemaphoreType.DMA,
        ],
    )(_sc_body)
    return kfn(dyn_idx, static_idx, time, data_tbl, static_tbl, time_w, time_b)


def kernel(dynamic_indices, static_indices, time, data_emb_table,
           static_emb_table, time_w, time_b):
    out = _run(dynamic_indices, static_indices, time.reshape(-1),
               data_emb_table, static_emb_table, time_w, time_b)
    return out.reshape(B, S, H)
